# Initial kernel scaffold; baseline (speedup 1.0000x reference)
import jax
import jax.numpy as jnp
from jax.experimental import pallas as pl


def kernel(x, edge_index, W, attn_l, attn_r, hop_attn_l, hop_attn_r):
    raise NotImplementedError("write your pallas kernel here")


# speedup vs baseline: 5.7267x; 5.7267x over previous
import jax
import jax.numpy as jnp
from jax import lax
from jax.experimental import pallas as pl
from jax.experimental.pallas import tpu as pltpu
from jax.experimental.pallas import tpu_sc as plsc

N = 10000
NP = 10240
E = 320000
E2 = 327680
D = 128
DH = 64
NEG = 0.2
RPT = 640

_MESH = plsc.VectorSubcoreMesh(
    core_axis_name="c", subcore_axis_name="s", num_cores=2, num_subcores=16)


_BLK = 400


def _tc_body(x_ref, w_ref, al_ref, ar_ref, h0_ref, h1_ref, el_ref, er_ref):
    h = lax.dot_general(x_ref[...], w_ref[...], (((1,), (1,)), ((), ())),
                        preferred_element_type=jnp.float32)
    h0_ref[...] = h[:, :DH]
    h1_ref[...] = h[:, DH:]
    el_ref[...] = jnp.sum(h * al_ref[...], axis=1, keepdims=True)
    er_ref[...] = jnp.sum(h * ar_ref[...], axis=1, keepdims=True)


def _tc_project(x, w, al, ar):
    return pl.pallas_call(
        _tc_body,
        grid=(N // _BLK,),
        in_specs=[
            pl.BlockSpec((_BLK, D), lambda i: (i, 0)),
            pl.BlockSpec((D, D), lambda i: (0, 0)),
            pl.BlockSpec((1, D), lambda i: (0, 0)),
            pl.BlockSpec((1, D), lambda i: (0, 0)),
        ],
        out_specs=[
            pl.BlockSpec((_BLK, DH), lambda i: (i, 0)),
            pl.BlockSpec((_BLK, DH), lambda i: (i, 0)),
            pl.BlockSpec((_BLK, 1), lambda i: (i, 0)),
            pl.BlockSpec((_BLK, 1), lambda i: (i, 0)),
        ],
        out_shape=[
            jax.ShapeDtypeStruct((N, DH), jnp.float32),
            jax.ShapeDtypeStruct((N, DH), jnp.float32),
            jax.ShapeDtypeStruct((N, 1), jnp.float32),
            jax.ShapeDtypeStruct((N, 1), jnp.float32),
        ],
    )(x, w, al, ar)


def _rsqrt_nr(xv):
    i = lax.bitcast_convert_type(xv, jnp.int32)
    i = jnp.int32(0x5F3759DF) - lax.shift_right_logical(i, 1)
    y = lax.bitcast_convert_type(i, jnp.float32)
    for _ in range(3):
        y = y * (1.5 - 0.5 * xv * y * y)
    return y


def _coef_body(el_hbm, er_hbm, src_hbm, dst_hbm, c_hbm,
               el_t, er_t, den_l, ind_l, out_l, qs_t, qd_t,
               tmp2, dn_r, in_r, ou_r, si_t, di_t, c_t,
               den_s, ind_s, out_s, qs_s, qd_s):
    cid = lax.axis_index("c")
    sid = lax.axis_index("s")
    wid = sid * 2 + cid
    zf = jnp.zeros((16,), jnp.float32)
    ones = jnp.ones((16,), jnp.float32)
    lanes = lax.iota(jnp.int32, 16)

    def zacc(i, carry):
        den_l[pl.ds(i * 16, 16)] = zf
        ind_l[pl.ds(i * 16, 16)] = zf
        out_l[pl.ds(i * 16, 16)] = zf
        return carry

    lax.fori_loop(0, NP // 16, zacc, 0)

    pltpu.sync_copy(el_hbm, el_t)
    pltpu.sync_copy(er_hbm, er_t)

    def p1_chunk(ci, carry):
        base = sid * (E2 // 16) + ci * 2048
        pltpu.sync_copy(src_hbm.at[pl.ds(base, 2048)], si_t)
        pltpu.sync_copy(dst_hbm.at[pl.ds(base, 2048)], di_t)

        def grp(g, c2):
            isv = si_t[pl.ds(g * 16, 16)]
            idv = di_t[pl.ds(g * 16, 16)]
            ev = plsc.load_gather(el_t, [isv]) + plsc.load_gather(er_t, [idv])
            ev = jnp.where(ev >= 0.0, ev, NEG * ev)
            ee = jnp.exp(ev)
            m = (base + g * 16 + lanes) < E
            plsc.addupdate_scatter(den_l, [idv], ee, mask=m)
            plsc.addupdate_scatter(ind_l, [idv], ones, mask=m)
            plsc.addupdate_scatter(out_l, [isv], ones, mask=m)
            return c2

        lax.fori_loop(0, 128, grp, 0)
        return carry

    lax.fori_loop(0, 10, p1_chunk, 0)

    pltpu.sync_copy(den_l, den_s.at[sid])
    pltpu.sync_copy(ind_l, ind_s.at[sid])
    pltpu.sync_copy(out_l, out_s.at[sid])
    plsc.subcore_barrier()

    off = sid * RPT

    def reduce16(part_s, red_t):
        pltpu.sync_copy(part_s.at[:, pl.ds(off, RPT)], tmp2)

        def rgrp(g, carry):
            s = pl.ds(g * 16, 16)
            acc = tmp2[0, s]
            for t in range(1, 16):
                acc = acc + tmp2[t, s]
            red_t[s] = acc
            return carry

        lax.fori_loop(0, RPT // 16, rgrp, 0)

    reduce16(den_s, dn_r)
    reduce16(ind_s, in_r)
    reduce16(out_s, ou_r)

    def qgrp(g, carry):
        s = pl.ds(g * 16, 16)
        qs = _rsqrt_nr(jnp.maximum(ou_r[s], 1.0))
        indc = jnp.maximum(in_r[s], 1.0)
        rden = _rsqrt_nr(jnp.maximum(dn_r[s], 1e-16))
        qd = indc * _rsqrt_nr(indc) * rden * rden
        qs_t[s] = qs
        qd_t[s] = qd
        return carry

    lax.fori_loop(0, RPT // 16, qgrp, 0)
    pltpu.sync_copy(qs_t.at[pl.ds(0, RPT)], qs_s.at[pl.ds(off, RPT)])
    pltpu.sync_copy(qd_t.at[pl.ds(0, RPT)], qd_s.at[pl.ds(off, RPT)])
    plsc.subcore_barrier()
    pltpu.sync_copy(qs_s, qs_t)
    pltpu.sync_copy(qd_s, qd_t)

    def p2_chunk(ci, carry):
        base = wid * (E2 // 32) + ci * 2048
        pltpu.sync_copy(src_hbm.at[pl.ds(base, 2048)], si_t)
        pltpu.sync_copy(dst_hbm.at[pl.ds(base, 2048)], di_t)

        def grp(g, c2):
            isv = si_t[pl.ds(g * 16, 16)]
            idv = di_t[pl.ds(g * 16, 16)]
            ev = plsc.load_gather(el_t, [isv]) + plsc.load_gather(er_t, [idv])
            ev = jnp.where(ev >= 0.0, ev, NEG * ev)
            ee = jnp.exp(ev)
            qsv = plsc.load_gather(qs_t, [isv])
            qdv = plsc.load_gather(qd_t, [idv])
            cv = ee * qsv * qdv
            eid = base + g * 16 + lanes
            c_t[pl.ds(g * 16, 16)] = jnp.where(eid < E, cv, 0.0)
            return c2

        lax.fori_loop(0, 128, grp, 0)
        pltpu.sync_copy(c_t, c_hbm.at[pl.ds(base, 2048)])
        return carry

    lax.fori_loop(0, 5, p2_chunk, 0)


def _sc_coef(el, er, srcp, dstp):
    f = pl.kernel(
        _coef_body,
        out_type=jax.ShapeDtypeStruct((E2,), jnp.float32),
        mesh=_MESH,
        compiler_params=pltpu.CompilerParams(needs_layout_passes=False),
        scratch_types=[
            pltpu.VMEM((N,), jnp.float32),
            pltpu.VMEM((N,), jnp.float32),
            pltpu.VMEM((NP,), jnp.float32),
            pltpu.VMEM((NP,), jnp.float32),
            pltpu.VMEM((NP,), jnp.float32),
            pltpu.VMEM((NP,), jnp.float32),
            pltpu.VMEM((NP,), jnp.float32),
            pltpu.VMEM((16, RPT), jnp.float32),
            pltpu.VMEM((RPT,), jnp.float32),
            pltpu.VMEM((RPT,), jnp.float32),
            pltpu.VMEM((RPT,), jnp.float32),
            pltpu.VMEM((2048,), jnp.int32),
            pltpu.VMEM((2048,), jnp.int32),
            pltpu.VMEM((2048,), jnp.float32),
            pltpu.VMEM_SHARED((16, NP), jnp.float32),
            pltpu.VMEM_SHARED((16, NP), jnp.float32),
            pltpu.VMEM_SHARED((16, NP), jnp.float32),
            pltpu.VMEM_SHARED((NP,), jnp.float32),
            pltpu.VMEM_SHARED((NP,), jnp.float32),
        ],
    )
    return f(el, er, srcp, dstp)


def _hop_body(h0_hbm, h1_hbm, src2_hbm, dst2_hbm, c_hbm,
              out0_hbm, out1_hbm, wa0_hbm, wa1_hbm, wb0_hbm, wb1_hbm,
              rows_t, si_t, di_t, cc_t, acc_s):
    cid = lax.axis_index("c")
    sid = lax.axis_index("s")
    r0 = sid * RPT
    zf = jnp.zeros((16,), jnp.float32)

    def zrows(g, carry):
        for j in range(4):
            rows_t[g, pl.ds(j * 16, 16)] = zf
        return carry

    tables = ((h0_hbm, h1_hbm), (wa0_hbm, wa1_hbm), (wb0_hbm, wb1_hbm))
    dumps = ((wa0_hbm, wa1_hbm), (wb0_hbm, wb1_hbm), (out0_hbm, out1_hbm))
    for hop in range(3):
        t0, t1 = tables[hop]
        u0, u1 = dumps[hop]

        lax.fori_loop(0, 128, zrows, 0)
        for k in range(5):
            pltpu.sync_copy(rows_t, acc_s.at[pl.ds(r0 + k * 128, 128), :])
        plsc.subcore_barrier()

        def chunk(ci, carry):
            rb = sid * 160 + ci * 8
            pltpu.sync_copy(src2_hbm.at[pl.ds(rb, 8), :], si_t)
            pltpu.sync_copy(dst2_hbm.at[pl.ds(rb, 8), :], di_t)
            pltpu.sync_copy(c_hbm.at[pl.ds(rb * 128, 1024)], cc_t)
            for j in range(8):
                @pl.when(cid == 0)
                def _():
                    pltpu.sync_copy(t0.at[si_t.at[j]], rows_t)

                @pl.when(cid == 1)
                def _():
                    pltpu.sync_copy(t1.at[si_t.at[j]], rows_t)

                def grp16(gg, c2):
                    cvec = cc_t[pl.ds(j * 128 + gg * 16, 16)]
                    for l in range(16):
                        g = gg * 16 + l
                        cv = cvec[l]
                        for q in range(4):
                            s = pl.ds(q * 16, 16)
                            rows_t[g, s] = rows_t[g, s] * cv
                    return c2

                lax.fori_loop(0, 8, grp16, 0)
                pltpu.sync_copy(rows_t, acc_s.at[di_t.at[j]], add=True)
            return carry

        lax.fori_loop(0, 20, chunk, 0)
        plsc.subcore_barrier()

        for k in range(5):
            rb2 = jnp.minimum(r0 + k * 128, N - 128)
            pltpu.sync_copy(acc_s.at[pl.ds(rb2, 128), :], rows_t)

            @pl.when(cid == 0)
            def _():
                pltpu.sync_copy(rows_t, u0.at[pl.ds(rb2, 128), :])

            @pl.when(cid == 1)
            def _():
                pltpu.sync_copy(rows_t, u1.at[pl.ds(rb2, 128), :])

        plsc.subcore_barrier()


def _sc_hops(h0, h1, src2, dst2, c):
    sds = jax.ShapeDtypeStruct((N, DH), jnp.float32)
    f = pl.kernel(
        _hop_body,
        out_type=[sds, sds, sds, sds, sds, sds],
        mesh=_MESH,
        compiler_params=pltpu.CompilerParams(
            needs_layout_passes=False, use_tc_tiling_on_sc=False),
        scratch_types=[
            pltpu.VMEM((128, DH), jnp.float32),
            pltpu.VMEM((8, 128), jnp.int32),
            pltpu.VMEM((8, 128), jnp.int32),
            pltpu.VMEM((1024,), jnp.float32),
            pltpu.VMEM_SHARED((NP, DH), jnp.float32),
        ],
    )
    out0, out1, _, _, _, _ = f(h0, h1, src2, dst2, c)
    return out0, out1


def kernel(x, edge_index, W, attn_l, attn_r, hop_attn_l, hop_attn_r):
    al = attn_l.reshape(1, D)
    ar = attn_r.reshape(1, D)
    h0, h1, el2, er2 = _tc_project(x, W, al, ar)
    srcp = jnp.pad(edge_index[0], (0, E2 - E))
    dstp = jnp.pad(edge_index[1], (0, E2 - E))
    c = _sc_coef(el2.reshape(-1), er2.reshape(-1), srcp, dstp)
    src2 = srcp.reshape(E2 // 128, 128)
    dst2 = dstp.reshape(E2 // 128, 128)
    out0, out1 = _sc_hops(h0, h1, src2, dst2, c)
    return jnp.concatenate([out0, out1], axis=1).reshape(N, 1, D)


# speedup vs baseline: 13.3215x; 2.3262x over previous
import jax
import jax.numpy as jnp
from jax import lax
from jax.experimental import pallas as pl
from jax.experimental.pallas import tpu as pltpu
from jax.experimental.pallas import tpu_sc as plsc

N = 10000
NP = 10240
E = 320000
E2 = 327680
D = 128
DH = 64
NEG = 0.2
RPT = 640

_MESH = plsc.VectorSubcoreMesh(
    core_axis_name="c", subcore_axis_name="s", num_cores=2, num_subcores=16)


_BLK = 400


def _tc_body(x_ref, w_ref, al_ref, ar_ref, h0_ref, h1_ref, el_ref, er_ref):
    h = lax.dot_general(x_ref[...], w_ref[...], (((1,), (1,)), ((), ())),
                        preferred_element_type=jnp.float32)
    h0_ref[...] = h[:, :DH]
    h1_ref[...] = h[:, DH:]
    el_ref[...] = jnp.sum(h * al_ref[...], axis=1, keepdims=True)
    er_ref[...] = jnp.sum(h * ar_ref[...], axis=1, keepdims=True)


def _tc_project(x, w, al, ar):
    return pl.pallas_call(
        _tc_body,
        grid=(N // _BLK,),
        in_specs=[
            pl.BlockSpec((_BLK, D), lambda i: (i, 0)),
            pl.BlockSpec((D, D), lambda i: (0, 0)),
            pl.BlockSpec((1, D), lambda i: (0, 0)),
            pl.BlockSpec((1, D), lambda i: (0, 0)),
        ],
        out_specs=[
            pl.BlockSpec((_BLK, DH), lambda i: (i, 0)),
            pl.BlockSpec((_BLK, DH), lambda i: (i, 0)),
            pl.BlockSpec((_BLK, 1), lambda i: (i, 0)),
            pl.BlockSpec((_BLK, 1), lambda i: (i, 0)),
        ],
        out_shape=[
            jax.ShapeDtypeStruct((N, DH), jnp.float32),
            jax.ShapeDtypeStruct((N, DH), jnp.float32),
            jax.ShapeDtypeStruct((N, 1), jnp.float32),
            jax.ShapeDtypeStruct((N, 1), jnp.float32),
        ],
    )(x, w, al, ar)


def _rsqrt_nr(xv):
    i = lax.bitcast_convert_type(xv, jnp.int32)
    i = jnp.int32(0x5F3759DF) - lax.shift_right_logical(i, 1)
    y = lax.bitcast_convert_type(i, jnp.float32)
    for _ in range(3):
        y = y * (1.5 - 0.5 * xv * y * y)
    return y


def _coef_body(el_hbm, er_hbm, src_hbm, dst_hbm, c_hbm,
               el_t, er_t, den_l, ind_l, out_l, qs_t, qd_t,
               tmp2, dn_r, in_r, ou_r, si_t, di_t, c_t,
               den_s, ind_s, out_s, qs_s, qd_s):
    cid = lax.axis_index("c")
    sid = lax.axis_index("s")
    wid = sid * 2 + cid
    zf = jnp.zeros((16,), jnp.float32)
    ones = jnp.ones((16,), jnp.float32)
    lanes = lax.iota(jnp.int32, 16)

    def zacc(i, carry):
        den_l[pl.ds(i * 16, 16)] = zf
        ind_l[pl.ds(i * 16, 16)] = zf
        out_l[pl.ds(i * 16, 16)] = zf
        return carry

    lax.fori_loop(0, NP // 16, zacc, 0)

    pltpu.sync_copy(el_hbm, el_t)
    pltpu.sync_copy(er_hbm, er_t)

    def p1_chunk(ci, carry):
        base = sid * (E2 // 16) + ci * 2048
        pltpu.sync_copy(src_hbm.at[pl.ds(base, 2048)], si_t)
        pltpu.sync_copy(dst_hbm.at[pl.ds(base, 2048)], di_t)

        def grp(g, c2):
            isv = si_t[pl.ds(g * 16, 16)]
            idv = di_t[pl.ds(g * 16, 16)]
            ev = plsc.load_gather(el_t, [isv]) + plsc.load_gather(er_t, [idv])
            ev = jnp.where(ev >= 0.0, ev, NEG * ev)
            ee = jnp.exp(ev)
            m = (base + g * 16 + lanes) < E
            plsc.addupdate_scatter(den_l, [idv], ee, mask=m)
            plsc.addupdate_scatter(ind_l, [idv], ones, mask=m)
            plsc.addupdate_scatter(out_l, [isv], ones, mask=m)
            return c2

        lax.fori_loop(0, 128, grp, 0)
        return carry

    lax.fori_loop(0, 10, p1_chunk, 0)

    pltpu.sync_copy(den_l, den_s.at[sid])
    pltpu.sync_copy(ind_l, ind_s.at[sid])
    pltpu.sync_copy(out_l, out_s.at[sid])
    plsc.subcore_barrier()

    off = sid * RPT

    def reduce16(part_s, red_t):
        pltpu.sync_copy(part_s.at[:, pl.ds(off, RPT)], tmp2)

        def rgrp(g, carry):
            s = pl.ds(g * 16, 16)
            acc = tmp2[0, s]
            for t in range(1, 16):
                acc = acc + tmp2[t, s]
            red_t[s] = acc
            return carry

        lax.fori_loop(0, RPT // 16, rgrp, 0)

    reduce16(den_s, dn_r)
    reduce16(ind_s, in_r)
    reduce16(out_s, ou_r)

    def qgrp(g, carry):
        s = pl.ds(g * 16, 16)
        qs = _rsqrt_nr(jnp.maximum(ou_r[s], 1.0))
        indc = jnp.maximum(in_r[s], 1.0)
        rden = _rsqrt_nr(jnp.maximum(dn_r[s], 1e-16))
        qd = indc * _rsqrt_nr(indc) * rden * rden
        qs_t[s] = qs
        qd_t[s] = qd
        return carry

    lax.fori_loop(0, RPT // 16, qgrp, 0)
    pltpu.sync_copy(qs_t.at[pl.ds(0, RPT)], qs_s.at[pl.ds(off, RPT)])
    pltpu.sync_copy(qd_t.at[pl.ds(0, RPT)], qd_s.at[pl.ds(off, RPT)])
    plsc.subcore_barrier()
    pltpu.sync_copy(qs_s, qs_t)
    pltpu.sync_copy(qd_s, qd_t)

    def p2_chunk(ci, carry):
        base = wid * (E2 // 32) + ci * 2048
        pltpu.sync_copy(src_hbm.at[pl.ds(base, 2048)], si_t)
        pltpu.sync_copy(dst_hbm.at[pl.ds(base, 2048)], di_t)

        def grp(g, c2):
            isv = si_t[pl.ds(g * 16, 16)]
            idv = di_t[pl.ds(g * 16, 16)]
            ev = plsc.load_gather(el_t, [isv]) + plsc.load_gather(er_t, [idv])
            ev = jnp.where(ev >= 0.0, ev, NEG * ev)
            ee = jnp.exp(ev)
            qsv = plsc.load_gather(qs_t, [isv])
            qdv = plsc.load_gather(qd_t, [idv])
            cv = ee * qsv * qdv
            eid = base + g * 16 + lanes
            c_t[pl.ds(g * 16, 16)] = jnp.where(eid < E, cv, 0.0)
            return c2

        lax.fori_loop(0, 128, grp, 0)
        pltpu.sync_copy(c_t, c_hbm.at[pl.ds(base, 2048)])
        return carry

    lax.fori_loop(0, 5, p2_chunk, 0)


def _sc_coef(el, er, srcp, dstp):
    f = pl.kernel(
        _coef_body,
        out_type=jax.ShapeDtypeStruct((E2,), jnp.float32),
        mesh=_MESH,
        compiler_params=pltpu.CompilerParams(needs_layout_passes=False),
        scratch_types=[
            pltpu.VMEM((N,), jnp.float32),
            pltpu.VMEM((N,), jnp.float32),
            pltpu.VMEM((NP,), jnp.float32),
            pltpu.VMEM((NP,), jnp.float32),
            pltpu.VMEM((NP,), jnp.float32),
            pltpu.VMEM((NP,), jnp.float32),
            pltpu.VMEM((NP,), jnp.float32),
            pltpu.VMEM((16, RPT), jnp.float32),
            pltpu.VMEM((RPT,), jnp.float32),
            pltpu.VMEM((RPT,), jnp.float32),
            pltpu.VMEM((RPT,), jnp.float32),
            pltpu.VMEM((2048,), jnp.int32),
            pltpu.VMEM((2048,), jnp.int32),
            pltpu.VMEM((2048,), jnp.float32),
            pltpu.VMEM_SHARED((16, NP), jnp.float32),
            pltpu.VMEM_SHARED((16, NP), jnp.float32),
            pltpu.VMEM_SHARED((16, NP), jnp.float32),
            pltpu.VMEM_SHARED((NP,), jnp.float32),
            pltpu.VMEM_SHARED((NP,), jnp.float32),
        ],
    )
    return f(el, er, srcp, dstp)


_NBUF = 8
_NBLK = 160
_NCH = 20


def _hop_body(h0_hbm, h1_hbm, src2_hbm, dst2_hbm, c_hbm,
              out0_hbm, out1_hbm, w0_hbm, w1_hbm,
              si0, si1, si2, di0, di1, di2, cc0, cc1, cc2,
              r0, r1, r2, r3, r4, r5, r6, r7,
              acc_s,
              g0, g1, g2, g3, g4, g5, g6, g7,
              s0, s1, s2, s3, s4, s5, s6, s7,
              m0, m1, m2):
    cid = lax.axis_index("c")
    sid = lax.axis_index("s")
    racc = sid * RPT
    zf = jnp.zeros((16,), jnp.float32)
    rbufs = (r0, r1, r2, r3, r4, r5, r6, r7)
    gsem = (g0, g1, g2, g3, g4, g5, g6, g7)
    ssem = (s0, s1, s2, s3, s4, s5, s6, s7)
    sib = (si0, si1, si2)
    dib = (di0, di1, di2)
    ccb = (cc0, cc1, cc2)
    msem = (m0, m1, m2)
    mrow = sid * _NBLK

    def zrows(g, carry):
        for j in range(4):
            r0[g, pl.ds(j * 16, 16)] = zf
        return carry

    def mstart(c, mb):
        pltpu.async_copy(src2_hbm.at[pl.ds(mrow + c * 8, 8), :], sib[mb],
                         msem[mb])
        pltpu.async_copy(dst2_hbm.at[pl.ds(mrow + c * 8, 8), :], dib[mb],
                         msem[mb])
        pltpu.async_copy(c_hbm.at[pl.ds((mrow + c * 8) * 128, 1024)],
                         ccb[mb].at[pl.ds(0, 1024)], msem[mb])

    def mwait(mb):
        pltpu.make_async_copy(src2_hbm.at[pl.ds(0, 8), :], sib[mb],
                              msem[mb]).wait()
        pltpu.make_async_copy(dst2_hbm.at[pl.ds(0, 8), :], dib[mb],
                              msem[mb]).wait()
        pltpu.make_async_copy(c_hbm.at[pl.ds(0, 1024)],
                              ccb[mb].at[pl.ds(0, 1024)], msem[mb]).wait()

    def scale(k, mb, rb):
        def row8(gg, c2):
            cvec = ccb[mb][pl.ds(k * 128 + gg * 8, 16)]
            for l in range(8):
                g = gg * 8 + l
                cv = cvec[l]
                for qd in range(4):
                    s = pl.ds(qd * 16, 16)
                    rb[g, s] = rb[g, s] * cv
            return c2

        lax.fori_loop(0, 16, row8, 0)

    for k in range(5):
        rb2 = jnp.minimum(racc + k * 128, N - 128)

        @pl.when(cid == 0)
        def _():
            pltpu.sync_copy(h0_hbm.at[pl.ds(rb2, 128), :], r0)
            pltpu.sync_copy(r0, w0_hbm.at[0].at[pl.ds(rb2, 128), :])

        @pl.when(cid == 1)
        def _():
            pltpu.sync_copy(h1_hbm.at[pl.ds(rb2, 128), :], r0)
            pltpu.sync_copy(r0, w1_hbm.at[0].at[pl.ds(rb2, 128), :])

    plsc.subcore_barrier()

    def hop_body(p, carry):
        q = p % 2

        def gstart(row, mb, b):
            @pl.when(cid == 0)
            def _():
                pltpu.async_copy(w0_hbm.at[q].at[sib[mb].at[row]],
                                 rbufs[b], gsem[b])

            @pl.when(cid == 1)
            def _():
                pltpu.async_copy(w1_hbm.at[q].at[sib[mb].at[row]],
                                 rbufs[b], gsem[b])

        def gwait(b):
            pltpu.make_async_copy(w0_hbm.at[0].at[si0.at[0]], rbufs[b],
                                  gsem[b]).wait()

        def swait(b):
            pltpu.make_async_copy(rbufs[b], acc_s.at[di0.at[0]],
                                  ssem[b]).wait()

        def chunk(c, mb, first, last):
            nmb = (mb + 1) % 3
            for k in range(8):
                if k == 0 and not last:
                    mstart(c + 1, nmb)
                if k == 4 and not last:
                    mwait(nmb)
                jslot = (k + 4) % 8
                if not last or k < 4:
                    if not first or k >= 4:
                        swait(jslot)
                    if k < 4:
                        gstart(k + 4, mb, jslot)
                    else:
                        gstart(k - 4, nmb, jslot)
                gwait(k)
                scale(k, mb, rbufs[k])
                pltpu.async_copy(rbufs[k], acc_s.at[dib[mb].at[k]], ssem[k],
                                 add=True)

        lax.fori_loop(0, 128, zrows, 0)
        for k in range(5):
            pltpu.sync_copy(r0, acc_s.at[pl.ds(racc + k * 128, 128), :])
        plsc.subcore_barrier()

        mstart(0, 0)
        mwait(0)
        for b in range(4):
            gstart(b, 0, b)

        chunk(0, 0, True, False)

        def triple(g, c2):
            cb3 = 1 + g * 3
            for h in range(3):
                chunk(cb3 + h, (1 + h) % 3, False, False)
            return c2

        lax.fori_loop(0, 6, triple, 0)
        chunk(_NCH - 1, (_NCH - 1) % 3, False, True)

        for b in range(_NBUF):
            swait(b)
        plsc.subcore_barrier()

        for k in range(5):
            rb2 = jnp.minimum(racc + k * 128, N - 128)
            pltpu.sync_copy(acc_s.at[pl.ds(rb2, 128), :], r0)

            @pl.when(cid == 0)
            def _():
                pltpu.sync_copy(r0, w0_hbm.at[1 - q].at[pl.ds(rb2, 128), :])

            @pl.when(cid == 1)
            def _():
                pltpu.sync_copy(r0, w1_hbm.at[1 - q].at[pl.ds(rb2, 128), :])

        plsc.subcore_barrier()
        return carry

    lax.fori_loop(0, 3, hop_body, 0)

    for k in range(5):
        rb2 = jnp.minimum(racc + k * 128, N - 128)

        @pl.when(cid == 0)
        def _():
            pltpu.sync_copy(w0_hbm.at[1].at[pl.ds(rb2, 128), :], r0)
            pltpu.sync_copy(r0, out0_hbm.at[pl.ds(rb2, 128), :])

        @pl.when(cid == 1)
        def _():
            pltpu.sync_copy(w1_hbm.at[1].at[pl.ds(rb2, 128), :], r0)
            pltpu.sync_copy(r0, out1_hbm.at[pl.ds(rb2, 128), :])


def _sc_hops(h0, h1, src2, dst2, c):
    sds = jax.ShapeDtypeStruct((N, DH), jnp.float32)
    wds = jax.ShapeDtypeStruct((2, N, DH), jnp.float32)
    f = pl.kernel(
        _hop_body,
        out_type=[sds, sds, wds, wds],
        mesh=_MESH,
        compiler_params=pltpu.CompilerParams(
            needs_layout_passes=False, use_tc_tiling_on_sc=False),
        scratch_types=(
            [pltpu.VMEM((8, 128), jnp.int32)] * 3
            + [pltpu.VMEM((8, 128), jnp.int32)] * 3
            + [pltpu.VMEM((1040,), jnp.float32)] * 3
            + [pltpu.VMEM((128, DH), jnp.float32)] * _NBUF
            + [pltpu.VMEM_SHARED((NP, DH), jnp.float32)]
            + [pltpu.SemaphoreType.DMA] * (2 * _NBUF)
            + [pltpu.SemaphoreType.DMA] * 3
        ),
    )
    out0, out1, _, _ = f(h0, h1, src2, dst2, c)
    return out0, out1


def kernel(x, edge_index, W, attn_l, attn_r, hop_attn_l, hop_attn_r):
    al = attn_l.reshape(1, D)
    ar = attn_r.reshape(1, D)
    h0, h1, el2, er2 = _tc_project(x, W, al, ar)
    srcp = jnp.pad(edge_index[0], (0, E2 - E))
    dstp = jnp.pad(edge_index[1], (0, E2 - E))
    c = _sc_coef(el2.reshape(-1), er2.reshape(-1), srcp, dstp)
    src2 = srcp.reshape(E2 // 128, 128)
    dst2 = dstp.reshape(E2 // 128, 128)
    out0, out1 = _sc_hops(h0, h1, src2, dst2, c)
    return jnp.concatenate([out0, out1], axis=1).reshape(N, 1, D)
